# baseline (speedup 1.0000x reference)
import functools

import jax
import jax.numpy as jnp
from jax import lax
from jax.experimental import pallas as pl
from jax.experimental.pallas import tpu as pltpu
from jax.experimental.pallas import tpu_sc as plsc

B = 16384
ND = 16
NC = 2
NS = 16
NW = NC * NS
BPW = B // NW
CHUNK = 128
NCHUNK = BPW // CHUNK
NBLK = BPW // ND


def _sc_body(user_hbm, item_hbm, uids_hbm, pids_hbm, nids_hbm, d_hbm,
             idx_u, idx_p, idx_n, u_rows, p_rows, n_rows, d_v, sem):
    wid = lax.axis_index("s") * NC + lax.axis_index("c")
    base = wid * BPW

    pltpu.sync_copy(uids_hbm.at[pl.ds(base, BPW)], idx_u)
    pltpu.sync_copy(pids_hbm.at[pl.ds(base, BPW)], idx_p)
    pltpu.sync_copy(nids_hbm.at[pl.ds(base, BPW)], idx_n)

    copies = []
    for k in range(NCHUNK):
        sl = pl.ds(k * CHUNK, CHUNK)
        copies.append(pltpu.async_copy(
            user_hbm.at[idx_u.at[sl]], u_rows.at[sl], sem))
        copies.append(pltpu.async_copy(
            item_hbm.at[idx_p.at[sl]], p_rows.at[sl], sem))
        copies.append(pltpu.async_copy(
            item_hbm.at[idx_n.at[sl]], n_rows.at[sl], sem))
    for c in copies:
        c.wait()

    lane = lax.iota(jnp.int32, ND)

    def blk(b, _):
        rows = b * ND + lane
        acc = jnp.zeros((ND,), jnp.float32)
        for j in range(ND):
            col = jnp.full((ND,), j, jnp.int32)
            u = plsc.load_gather(u_rows, [rows, col])
            p = plsc.load_gather(p_rows, [rows, col])
            n = plsc.load_gather(n_rows, [rows, col])
            acc = acc + u * (p - n)
        d_v[pl.ds(b * ND, ND)] = acc
        return ()

    lax.fori_loop(0, NBLK, blk, (), unroll=False)

    pltpu.sync_copy(d_v, d_hbm.at[pl.ds(base, BPW)])


@jax.jit
def _sc_scores(user_emb, item_emb, uids, pids, nids):
    mesh = plsc.VectorSubcoreMesh(core_axis_name="c", subcore_axis_name="s")
    kfn = pl.kernel(
        _sc_body,
        out_type=jax.ShapeDtypeStruct((B,), jnp.float32),
        mesh=mesh,
        scratch_types=[
            pltpu.VMEM((BPW,), jnp.int32),
            pltpu.VMEM((BPW,), jnp.int32),
            pltpu.VMEM((BPW,), jnp.int32),
            pltpu.VMEM((BPW, ND), jnp.float32),
            pltpu.VMEM((BPW, ND), jnp.float32),
            pltpu.VMEM((BPW, ND), jnp.float32),
            pltpu.VMEM((BPW,), jnp.float32),
            pltpu.SemaphoreType.DMA,
        ],
        compiler_params=pltpu.CompilerParams(
            needs_layout_passes=False, use_tc_tiling_on_sc=False),
    )
    return kfn(user_emb, item_emb, uids, pids, nids)


def _loss_body(d_ref, out_ref):
    d = d_ref[...]
    loss = jnp.maximum(-d, 0.0) + jnp.log1p(jnp.exp(-jnp.abs(d)))
    out_ref[0, 0] = jnp.sum(loss) * (1.0 / B)


@jax.jit
def _tc_loss(d):
    return pl.pallas_call(
        _loss_body,
        out_shape=jax.ShapeDtypeStruct((1, 1), jnp.float32),
        out_specs=pl.BlockSpec(memory_space=pltpu.SMEM),
    )(d)


def kernel(X, user_emb, item_emb):
    uids = X[:, 0]
    pids = X[:, 1]
    nids = X[:, 2]
    d = _sc_scores(user_emb, item_emb, uids, pids, nids)
    loss = _tc_loss(d.reshape(128, 128))
    return loss.reshape(())
